# baseline (speedup 1.0000x reference)
import math

import jax
import jax.numpy as jnp
from jax import lax
from jax.experimental import pallas as pl
from jax.experimental.pallas import tpu as pltpu
from jax.experimental.pallas import tpu_sc as plsc

VOCAB = 1000000
D = 64
DP = 128
BATCH = 4096
SEQ = 200
SCALE = math.sqrt(D)

NC = 2
NS = 16
NW = NC * NS
ROWS_PER_W = BATCH // NW
IDX_BLOCK = 64

PAD_BLOCK = 5000


def _pad_kernel(t_ref, o_ref):
    o_ref[:, 0:D] = t_ref[...]


def _pad_table(table):
    return pl.pallas_call(
        _pad_kernel,
        grid=(VOCAB // PAD_BLOCK,),
        in_specs=[pl.BlockSpec((PAD_BLOCK, D), lambda i: (i, 0))],
        out_specs=pl.BlockSpec((PAD_BLOCK, DP), lambda i: (i, 0)),
        out_shape=jax.ShapeDtypeStruct((VOCAB, DP), jnp.float32),
    )(table)


def _embed_kernel(x_hbm, tpad_hbm, out_hbm,
                  idx_v, rows_v0, rows_v1, cbuf0, cbuf1,
                  gsem0, gsem1, osem0, osem1):
    wid = lax.axis_index("s") * NC + lax.axis_index("c")
    b0 = wid * ROWS_PER_W

    def stage_idx(r):
        off = pl.multiple_of(b0 + r, IDX_BLOCK)
        pltpu.sync_copy(x_hbm.at[pl.ds(off, IDX_BLOCK)], idx_v)

    def fire_gathers(r, rows_vb, gsemb):
        rr = lax.rem(r, IDX_BLOCK)
        pltpu.async_copy(tpad_hbm.at[idx_v.at[rr, pl.ds(0, 128)]],
                         rows_vb.at[pl.ds(0, 128)], gsemb)
        pltpu.async_copy(tpad_hbm.at[idx_v.at[rr, pl.ds(128, 72)]],
                         rows_vb.at[pl.ds(128, 72)], gsemb)

    def wait_gathers(rows_vb, gsemb):
        pltpu.make_async_copy(tpad_hbm.at[pl.ds(0, SEQ)], rows_vb, gsemb).wait()

    def scale_compact(rows_vb, cbufb):
        def srow(r2, _):
            for j in range(D // 16):
                sl = pl.ds(j * 16, 16)
                cbufb[r2, sl] = rows_vb[r2, sl] * SCALE
            return 0
        lax.fori_loop(0, SEQ, srow, 0, unroll=4)

    def fire_store(r, cbufb, osemb):
        pltpu.async_copy(cbufb, out_hbm.at[b0 + r], osemb)

    def wait_store(cbufb, osemb):
        pltpu.make_async_copy(cbufb, out_hbm.at[0], osemb).wait()

    stage_idx(0)
    fire_gathers(0, rows_v0, gsem0)

    def pair_body(g, _):
        for b in range(2):
            r = 2 * g + b
            if b == 0:
                cur_rows, cur_c, cur_g, cur_o = rows_v0, cbuf0, gsem0, osem0
                nxt_rows, nxt_c, nxt_g, nxt_o = rows_v1, cbuf1, gsem1, osem1
            else:
                cur_rows, cur_c, cur_g, cur_o = rows_v1, cbuf1, gsem1, osem1
                nxt_rows, nxt_c, nxt_g, nxt_o = rows_v0, cbuf0, gsem0, osem0

            wait_gathers(cur_rows, cur_g)

            @pl.when(r + 1 < ROWS_PER_W)
            def _():
                @pl.when(lax.rem(r + 1, IDX_BLOCK) == 0)
                def _():
                    stage_idx(r + 1)

                @pl.when(r >= 1)
                def _():
                    wait_store(nxt_c, nxt_o)
                fire_gathers(r + 1, nxt_rows, nxt_g)

            scale_compact(cur_rows, cur_c)
            fire_store(r, cur_c, cur_o)
        return 0

    lax.fori_loop(0, ROWS_PER_W // 2, pair_body, 0)
    wait_store(cbuf0, osem0)
    wait_store(cbuf1, osem1)


@jax.jit
def kernel(x, table):
    tpad = _pad_table(table)
    xi = x.astype(jnp.int32)
    mesh = plsc.VectorSubcoreMesh(
        core_axis_name="c", subcore_axis_name="s", num_cores=NC, num_subcores=NS
    )
    return pl.kernel(
        _embed_kernel,
        out_type=jax.ShapeDtypeStruct((BATCH, SEQ, D), jnp.float32),
        mesh=mesh,
        scratch_types=[
            pltpu.VMEM((IDX_BLOCK, SEQ), jnp.int32),
            pltpu.VMEM((SEQ, DP), jnp.float32),
            pltpu.VMEM((SEQ, DP), jnp.float32),
            pltpu.VMEM((SEQ, D), jnp.float32),
            pltpu.VMEM((SEQ, D), jnp.float32),
            pltpu.SemaphoreType.DMA,
            pltpu.SemaphoreType.DMA,
            pltpu.SemaphoreType.DMA,
            pltpu.SemaphoreType.DMA,
        ],
    )(xi, tpad)
